# baseline (speedup 1.0000x reference)
import functools

import jax
import jax.numpy as jnp
from jax import lax
from jax.experimental import pallas as pl
from jax.experimental.pallas import tpu as pltpu
from jax.experimental.pallas import tpu_sc as plsc

EMBED_DIM = 300
DE = 304
DO = 320
TOTAL = 4096 * 200
NPAIRS = TOTAL // 2
NUM_WORKERS = 32
PAIRS_PER_W = NPAIRS // NUM_WORKERS
NP = 64
NCHUNKS = PAIRS_PER_W // NP
HALF = NCHUNKS // 2

_mesh = plsc.VectorSubcoreMesh(core_axis_name="c", subcore_axis_name="s")


@functools.partial(
    pl.kernel,
    out_type=jax.ShapeDtypeStruct((NPAIRS, 2 * EMBED_DIM), jnp.float32),
    mesh=_mesh,
    scratch_types=[
        pltpu.VMEM((NCHUNKS, NP), jnp.int32),
        pltpu.VMEM((NCHUNKS, NP), jnp.int32),
        pltpu.VMEM((NP, DE), jnp.float32),
        pltpu.VMEM((NP, DE), jnp.float32),
        pltpu.VMEM((NP, DO), jnp.float32),
        pltpu.VMEM((NP, DO), jnp.float32),
        pltpu.VMEM((NP, 16), jnp.float32),
        pltpu.VMEM((NP, 16), jnp.float32),
        pltpu.SemaphoreType.DMA,
        pltpu.SemaphoreType.DMA,
        pltpu.SemaphoreType.DMA,
        pltpu.SemaphoreType.DMA,
        pltpu.SemaphoreType.DMA,
        pltpu.SemaphoreType.DMA,
        pltpu.SemaphoreType.DMA,
        pltpu.SemaphoreType.DMA,
        pltpu.SemaphoreType.DMA,
        pltpu.SemaphoreType.DMA,
    ],
    compiler_params=pltpu.CompilerParams(use_tc_tiling_on_sc=False),
)
def _embedding_gather(idxe_hbm, idxo_hbm, te_hbm, to_hbm, out_hbm,
                      idxe_v, idxo_v, buf_e0, buf_e1, buf_o0, buf_o1,
                      fix0, fix1,
                      sge0, sge1, sgo0, sgo1,
                      swe0, swe1, swo0, swo1, swf0, swf1):
    wid = lax.axis_index("s") * 2 + lax.axis_index("c")
    wrow = wid * NCHUNKS
    wpair = wid * PAIRS_PER_W

    pltpu.sync_copy(idxe_hbm.at[pl.ds(wrow, NCHUNKS)], idxe_v)
    pltpu.sync_copy(idxo_hbm.at[pl.ds(wrow, NCHUNKS)], idxo_v)

    buf_e = (buf_e0, buf_e1)
    buf_o = (buf_o0, buf_o1)
    fix = (fix0, fix1)
    sge = (sge0, sge1)
    sgo = (sgo0, sgo1)
    swe = (swe0, swe1)
    swo = (swo0, swo1)
    swf = (swf0, swf1)

    lane = lax.iota(jnp.int32, 16)
    mask = lane < 12

    def start(s, j):
        pltpu.async_copy(te_hbm.at[idxe_v.at[j]], buf_e[s], sge[s])
        pltpu.async_copy(to_hbm.at[idxo_v.at[j]], buf_o[s], sgo[s])

    def finish(s, j):
        pltpu.make_async_copy(te_hbm.at[idxe_v.at[j]], buf_e[s], sge[s]).wait()
        pltpu.make_async_copy(to_hbm.at[idxo_v.at[j]], buf_o[s], sgo[s]).wait()
        for k in range(NP):
            ve = buf_e[s].at[k][pl.ds(288, 16)]
            vo = buf_o[s].at[k][pl.ds(0, 16)]
            fix[s].at[k][...] = jnp.where(mask, ve, vo)
        dst = out_hbm.at[pl.ds(wpair + j * NP, NP)]
        we = pltpu.async_copy(buf_e[s].at[:, pl.ds(0, 288)], dst.at[:, pl.ds(0, 288)], swe[s])
        wo = pltpu.async_copy(buf_o[s].at[:, pl.ds(16, 296)], dst.at[:, pl.ds(304, 296)], swo[s])
        wf = pltpu.async_copy(fix[s], dst.at[:, pl.ds(288, 16)], swf[s])
        we.wait()
        wo.wait()
        wf.wait()

    start(0, 0)

    def body(t, carry):
        start(1, 2 * t + 1)
        finish(0, 2 * t)
        start(0, 2 * t + 2)
        finish(1, 2 * t + 1)
        return carry

    lax.fori_loop(0, HALF - 1, body, 0)

    t_last = HALF - 1
    start(1, 2 * t_last + 1)
    finish(0, 2 * t_last)
    finish(1, 2 * t_last + 1)


def kernel(indices, table):
    flat = indices.reshape(-1).astype(jnp.int32)
    idx_e = flat[0::2].reshape(NUM_WORKERS * NCHUNKS, NP)
    idx_o = flat[1::2].reshape(NUM_WORKERS * NCHUNKS, NP)
    t_e = jnp.pad(table, ((0, 0), (0, DE - EMBED_DIM)))
    t_o = jnp.pad(table, ((0, 0), (12, DO - EMBED_DIM - 12)))
    out = _embedding_gather(idx_e, idx_o, t_e, t_o)
    return out.reshape(indices.shape + (EMBED_DIM,))


# speedup vs baseline: 1.0697x; 1.0697x over previous
import functools

import jax
import jax.numpy as jnp
from jax import lax
from jax.experimental import pallas as pl
from jax.experimental.pallas import tpu as pltpu
from jax.experimental.pallas import tpu_sc as plsc

EMBED_DIM = 300
DP = 304
BATCH = 4096
SEQ = 200
TOTAL = BATCH * SEQ
NUM_WORKERS = 32
PER_WORKER = TOTAL // NUM_WORKERS
CH = 50
CPS = SEQ // CH
NCHUNKS = PER_WORKER // CH
HALF = NCHUNKS // 2

_mesh = plsc.VectorSubcoreMesh(core_axis_name="c", subcore_axis_name="s")


@functools.partial(
    pl.kernel,
    out_type=jax.ShapeDtypeStruct((BATCH, SEQ, EMBED_DIM), jnp.float32),
    mesh=_mesh,
    scratch_types=[
        pltpu.VMEM((NCHUNKS, CH), jnp.int32),
        pltpu.VMEM((CH, DP), jnp.float32),
        pltpu.VMEM((CH, DP), jnp.float32),
        pltpu.VMEM((CH, EMBED_DIM), jnp.float32),
        pltpu.VMEM((CH, EMBED_DIM), jnp.float32),
        pltpu.SemaphoreType.DMA,
        pltpu.SemaphoreType.DMA,
        pltpu.SemaphoreType.DMA,
        pltpu.SemaphoreType.DMA,
    ],
    compiler_params=pltpu.CompilerParams(use_tc_tiling_on_sc=False),
)
def _embedding_gather(idx_hbm, table_hbm, out_hbm,
                      idx_v, buf0, buf1, rows0, rows1,
                      sg0, sg1, sw0, sw1):
    wid = lax.axis_index("s") * 2 + lax.axis_index("c")
    wrow = wid * NCHUNKS
    wsample = wid * (PER_WORKER // SEQ)

    pltpu.sync_copy(idx_hbm.at[pl.ds(wrow, NCHUNKS)], idx_v)

    buf = (buf0, buf1)
    rows = (rows0, rows1)
    sg = (sg0, sg1)
    sw = (sw0, sw1)

    def start(s, j):
        pltpu.async_copy(table_hbm.at[idx_v.at[j]], buf[s], sg[s])

    def finish(s, j):
        pltpu.make_async_copy(table_hbm.at[idx_v.at[j]], buf[s], sg[s]).wait()

        def repack(r, carry):
            src = buf[s].at[r]
            dstr = rows[s].at[r]
            for c in range(0, EMBED_DIM - 16, 16):
                dstr[pl.ds(c, 16)] = src[pl.ds(c, 16)]
            dstr[pl.ds(EMBED_DIM - 16, 16)] = src[pl.ds(EMBED_DIM - 16, 16)]
            return carry

        lax.fori_loop(0, CH, repack, 0)
        bb = wsample + j // CPS
        t0 = (j % CPS) * CH
        pltpu.async_copy(rows[s], out_hbm.at[bb, pl.ds(t0, CH)], sw[s]).wait()

    start(0, 0)

    def body(t, carry):
        start(1, 2 * t + 1)
        finish(0, 2 * t)
        start(0, 2 * t + 2)
        finish(1, 2 * t + 1)
        return carry

    lax.fori_loop(0, HALF - 1, body, 0)

    t_last = HALF - 1
    start(1, 2 * t_last + 1)
    finish(0, 2 * t_last)
    finish(1, 2 * t_last + 1)


def kernel(indices, table):
    flat = indices.reshape(-1).astype(jnp.int32).reshape(TOTAL // CH, CH)
    table_p = jnp.pad(table, ((0, 0), (0, DP - EMBED_DIM)))
    return _embedding_gather(flat, table_p)
